# baseline (speedup 1.0000x reference)
import jax
import jax.numpy as jnp
from jax import lax
from jax.experimental import pallas as pl
from jax.experimental.pallas import tpu as pltpu
from jax.experimental.pallas import tpu_sc as plsc

NU = 10000
NE = 160000
FIN = 256
FH = 256
FO = 128
HF = 128

NC = 2
NS = 16
CH = 128
CPT = 80
HCPT = CPT // 2
NEP = NS * CPT * CH
DUMMY = NU
AGR = NU + 8
NPAD = 79 * 128
RB = 80
NRB = NU // RB


def _mesh():
    return plsc.VectorSubcoreMesh(
        core_axis_name="c", subcore_axis_name="s",
        num_cores=NC, num_subcores=NS)


def _deg_body(idx2, deg, idxv, onesv, zb, ob, hist):
    c = lax.axis_index("c")
    s = lax.axis_index("s")
    one = jnp.ones((16,), jnp.float32)
    zero = jnp.zeros((16,), jnp.float32)
    for k in range(CH // 16):
        onesv[pl.ds(16 * k, 16)] = one
    for k in range(RB // 16):
        zb[pl.ds(16 * k, 16)] = zero
    nblk = (NRB - s + NS - 1) // NS

    def zblk(k, carry):
        j = s + k * NS
        pltpu.sync_copy(zb, hist.at[pl.ds(j * RB, RB)])
        return carry

    lax.fori_loop(0, nblk, zblk, None)

    @pl.when(s == 0)
    def _():
        pltpu.sync_copy(zb.at[pl.ds(0, 8)], hist.at[pl.ds(NU, 8)])

    plsc.subcore_barrier()
    pltpu.sync_copy(idx2.at[c, s], idxv)

    def acc(j, carry):
        pltpu.sync_copy(onesv, hist.at[idxv.at[j]], add=True)
        return carry

    lax.fori_loop(0, CPT, acc, None)
    plsc.subcore_barrier()

    def dump(k, carry):
        j = s + k * NS
        pltpu.sync_copy(hist.at[pl.ds(j * RB, RB)], ob)
        pltpu.sync_copy(ob, deg.at[pl.ds(c * NU + j * RB, RB)])
        return carry

    lax.fori_loop(0, nblk, dump, None)


def _degrees(idx2):
    return pl.kernel(
        _deg_body,
        out_type=jax.ShapeDtypeStruct((2 * NU,), jnp.float32),
        mesh=_mesh(),
        scratch_types=[
            pltpu.VMEM((CPT, CH), jnp.int32),
            pltpu.VMEM((CH,), jnp.float32),
            pltpu.VMEM((RB,), jnp.float32),
            pltpu.VMEM((RB,), jnp.float32),
            pltpu.VMEM_SHARED((AGR,), jnp.float32),
        ],
    )(idx2)


def _zero_agg(buf, agg, s):
    zero = jnp.zeros((16,), jnp.float32)

    def zrow(i, carry):
        for k in range(8):
            buf[i, pl.ds(16 * k, 16)] = zero
        return carry

    lax.fori_loop(0, RB, zrow, None)
    nblk = (NRB - s + NS - 1) // NS

    def zblk(k, carry):
        j = s + k * NS
        pltpu.sync_copy(buf.at[pl.ds(0, RB)], agg.at[pl.ds(j * RB, RB)])
        return carry

    lax.fori_loop(0, nblk, zblk, None)

    @pl.when(s == 0)
    def _():
        pltpu.sync_copy(buf.at[pl.ds(0, 8)], agg.at[pl.ds(NU, 8)])


def _conv_pass(tab, gidx, sidx, buf, agg, out_slot, s):
    _zero_agg(buf, agg, s)
    plsc.subcore_barrier()

    def conv(j, carry):
        pltpu.sync_copy(tab.at[gidx.at[j]], buf)
        pltpu.sync_copy(buf, agg.at[sidx.at[j]], add=True)
        return carry

    lax.fori_loop(0, CPT, conv, None)
    plsc.subcore_barrier()
    nblk = (NRB - s + NS - 1) // NS

    def dump(k, carry):
        j = s + k * NS
        pltpu.sync_copy(agg.at[pl.ds(j * RB, RB)], buf.at[pl.ds(0, RB)])
        pltpu.sync_copy(buf.at[pl.ds(0, RB)], out_slot.at[pl.ds(j * RB, RB)])
        return carry

    lax.fori_loop(0, nblk, dump, None)
    plsc.subcore_barrier()


def _l1_body(hr3, hv3, idx2, agg4, srcv, dstv, buf, agg):
    c = lax.axis_index("c")
    s = lax.axis_index("s")
    pltpu.sync_copy(idx2.at[0, s], srcv)
    pltpu.sync_copy(idx2.at[1, s], dstv)
    for slot, tab3, gidx, sidx in ((0, hr3, srcv, dstv), (1, hv3, dstv, srcv)):
        _conv_pass(tab3.at[c], gidx, sidx, buf, agg, agg4.at[slot, c], s)


def _conv1(hr3, hv3, idx2):
    return pl.kernel(
        _l1_body,
        out_type=jax.ShapeDtypeStruct((2, 2, NU, HF), jnp.float32),
        mesh=_mesh(),
        scratch_types=[
            pltpu.VMEM((CPT, CH), jnp.int32),
            pltpu.VMEM((CPT, CH), jnp.int32),
            pltpu.VMEM((CH, HF), jnp.float32),
            pltpu.VMEM_SHARED((AGR, HF), jnp.float32),
        ],
    )(hr3, hv3, idx2)


def _l2_body(g3, idx2, norm2, b2f, out2, srcv, dstv, buf, nb, bb, agg):
    c = lax.axis_index("c")
    s = lax.axis_index("s")
    pltpu.sync_copy(idx2.at[c, s], srcv)
    pltpu.sync_copy(idx2.at[1 - c, s], dstv)
    pltpu.sync_copy(b2f.at[pl.ds(c * FO, FO)], bb)
    _zero_agg(buf, agg, s)
    plsc.subcore_barrier()

    def conv(j, carry):
        pltpu.sync_copy(g3.at[c].at[srcv.at[j]], buf)
        pltpu.sync_copy(buf, agg.at[dstv.at[j]], add=True)
        return carry

    lax.fori_loop(0, CPT, conv, None)
    plsc.subcore_barrier()
    nblk = (NRB - s + NS - 1) // NS

    def dump(k, carry):
        j = s + k * NS
        pltpu.sync_copy(agg.at[pl.ds(j * RB, RB)], buf.at[pl.ds(0, RB)])
        pltpu.sync_copy(norm2.at[pl.ds((1 - c) * NU + j * RB, RB)], nb)

        def grp(m, carry2):
            nv = nb[pl.ds(16 * m, 16)]
            for r in range(16):
                i = 16 * m + r
                bv = jnp.take(nv, jnp.full((16,), r, jnp.int32))
                for k2 in range(8):
                    sl = pl.ds(16 * k2, 16)
                    buf[i, sl] = buf[i, sl] * bv + bb[sl]
            return carry2

        lax.fori_loop(0, RB // 16, grp, None)
        pltpu.sync_copy(buf.at[pl.ds(0, RB)], out2.at[c, pl.ds(j * RB, RB)])
        return carry

    lax.fori_loop(0, nblk, dump, None)


def _conv2(g3, idx2, norm2, b2f):
    return pl.kernel(
        _l2_body,
        out_type=jax.ShapeDtypeStruct((2, NU, FO), jnp.float32),
        mesh=_mesh(),
        scratch_types=[
            pltpu.VMEM((CPT, CH), jnp.int32),
            pltpu.VMEM((CPT, CH), jnp.int32),
            pltpu.VMEM((CH, FO), jnp.float32),
            pltpu.VMEM((RB,), jnp.float32),
            pltpu.VMEM((FO,), jnp.float32),
            pltpu.VMEM_SHARED((AGR, FO), jnp.float32),
        ],
    )(g3, idx2, norm2, b2f)


_RBLK = 128
_NB = NPAD // _RBLK


def _mm1_body(xu, xi, du, di, wr, wv, hr3, hv3, norm3):
    h = pl.program_id(1)
    nu = lax.rsqrt(jnp.maximum(du[0], 1.0))
    ni = lax.rsqrt(jnp.maximum(di[0], 1.0))
    hr3[...] = jnp.dot(xu[...] * nu, wr[0],
                       preferred_element_type=jnp.float32)[None]
    hv3[...] = jnp.dot(xi[...] * ni, wv[0],
                       preferred_element_type=jnp.float32)[None]
    norm3[...] = jnp.where(h == 0, nu, ni)[None]


def _mm1(xu, xi, deg3, wr, wv):
    bs_x = pl.BlockSpec((_RBLK, FIN), lambda i, h: (i, 0))
    bs_du = pl.BlockSpec((1, _RBLK, 1), lambda i, h: (0, i, 0))
    bs_di = pl.BlockSpec((1, _RBLK, 1), lambda i, h: (1, i, 0))
    bs_w = pl.BlockSpec((1, FIN, HF), lambda i, h: (h, 0, 0))
    bs_h = pl.BlockSpec((1, _RBLK, HF), lambda i, h: (h, i, 0))
    bs_n = pl.BlockSpec((1, _RBLK, 1), lambda i, h: (h, i, 0))
    return pl.pallas_call(
        _mm1_body, grid=(_NB, 2),
        in_specs=[bs_x, bs_x, bs_du, bs_di, bs_w, bs_w],
        out_specs=[bs_h, bs_h, bs_n],
        out_shape=[jax.ShapeDtypeStruct((2, NPAD, HF), jnp.float32)] * 2
        + [jax.ShapeDtypeStruct((2, NU, 1), jnp.float32)],
    )(xu, xi, deg3, deg3, wr, wv)


def _mm2_body(a0, a1, dg, b1, w2, g3):
    n = lax.rsqrt(jnp.maximum(dg[0], 1.0))
    a = jnp.concatenate([a0[0, 0], a1[0, 0]], axis=1)
    t = jnp.maximum(a * n + b1[0], 0.0) * n
    g3[...] = jnp.dot(t, w2[0], preferred_element_type=jnp.float32)[None]


def _mm2(agg4, deg3, b1s, w2s):
    def bs_ak(k):
        return pl.BlockSpec((1, 1, _RBLK, HF),
                            lambda i, r, k=k: (1 - r, k, i, 0))
    bs_d = pl.BlockSpec((1, _RBLK, 1), lambda i, r: (r, i, 0))
    bs_b = pl.BlockSpec((1, 1, FH), lambda i, r: (r, 0, 0))
    bs_w = pl.BlockSpec((1, FH, FO), lambda i, r: (r, 0, 0))
    bs_g = pl.BlockSpec((1, _RBLK, FO), lambda i, r: (r, i, 0))
    return pl.pallas_call(
        _mm2_body, grid=(_NB, 2),
        in_specs=[bs_ak(0), bs_ak(1), bs_d, bs_b, bs_w],
        out_specs=bs_g,
        out_shape=jax.ShapeDtypeStruct((2, NPAD, FO), jnp.float32),
    )(agg4, agg4, deg3, b1s, w2s)


def kernel(x_user, x_item, src_rates, dst_rates,
           W1_rates, b1_rates, W1_rev, b1_rev,
           W2_rates, b2_rates, W2_rev, b2_rev):
    pad = jnp.full((NEP - NE,), DUMMY, jnp.int32)
    srcp = jnp.concatenate([src_rates, pad]).reshape(NS, CPT, CH)
    dstp = jnp.concatenate([dst_rates, pad]).reshape(NS, CPT, CH)
    idx2 = jnp.stack([srcp, dstp])
    deg = _degrees(idx2)
    deg3 = deg.reshape(2, NU, 1)
    w1rh = W1_rates.reshape(FIN, 2, HF).transpose(1, 0, 2)
    w1vh = W1_rev.reshape(FIN, 2, HF).transpose(1, 0, 2)
    hr3, hv3, norm3 = _mm1(x_user, x_item, deg3, w1rh, w1vh)
    norm2 = norm3.reshape(2 * NU)
    agg4 = _conv1(hr3, hv3, idx2)
    b1s = jnp.stack([b1_rev, b1_rates]).reshape(2, 1, FH)
    w2s = jnp.stack([W2_rates, W2_rev])
    g3 = _mm2(agg4, deg3, b1s, w2s)
    b2f = jnp.concatenate([b2_rates, b2_rev])
    h2 = _conv2(g3, idx2, norm2, b2f)
    return (h2[1], h2[0])


# speedup vs baseline: 1.6407x; 1.6407x over previous
import jax
import jax.numpy as jnp
from jax import lax
from jax.experimental import pallas as pl
from jax.experimental.pallas import tpu as pltpu
from jax.experimental.pallas import tpu_sc as plsc

NU = 10000
NE = 160000
FIN = 256
FH = 256
FO = 128
HF = 128

NC = 2
NS = 16
CH = 128
CPT = 79
NEP = NS * CPT * CH
DUMMY = NU
AGR = NU + 8
NPAD = 79 * 128
RB = 80
NRB = NU // RB


def _mesh():
    return plsc.VectorSubcoreMesh(
        core_axis_name="c", subcore_axis_name="s",
        num_cores=NC, num_subcores=NS)


def _deg_body(idx2, deg, idxv, onesv, zb, ob, hist):
    c = lax.axis_index("c")
    s = lax.axis_index("s")
    one = jnp.ones((16,), jnp.float32)
    zero = jnp.zeros((16,), jnp.float32)
    for k in range(CH // 16):
        onesv[pl.ds(16 * k, 16)] = one
    for k in range(RB // 16):
        zb[pl.ds(16 * k, 16)] = zero
    nblk = (NRB - s + NS - 1) // NS

    def zblk(k, carry):
        j = s + k * NS
        pltpu.sync_copy(zb, hist.at[pl.ds(j * RB, RB)])
        return carry

    lax.fori_loop(0, nblk, zblk, None)

    @pl.when(s == 0)
    def _():
        pltpu.sync_copy(zb.at[pl.ds(0, 8)], hist.at[pl.ds(NU, 8)])

    plsc.subcore_barrier()
    pltpu.sync_copy(idx2.at[c, s], idxv)

    def acc(j, carry):
        pltpu.sync_copy(onesv, hist.at[idxv.at[j]], add=True)
        return carry

    lax.fori_loop(0, CPT, acc, None)
    plsc.subcore_barrier()

    def dump(k, carry):
        j = s + k * NS
        pltpu.sync_copy(hist.at[pl.ds(j * RB, RB)], ob)
        pltpu.sync_copy(ob, deg.at[pl.ds(c * NU + j * RB, RB)])
        return carry

    lax.fori_loop(0, nblk, dump, None)


def _degrees(idx2):
    return pl.kernel(
        _deg_body,
        out_type=jax.ShapeDtypeStruct((2 * NU,), jnp.float32),
        mesh=_mesh(),
        scratch_types=[
            pltpu.VMEM((CPT, CH), jnp.int32),
            pltpu.VMEM((CH,), jnp.float32),
            pltpu.VMEM((RB,), jnp.float32),
            pltpu.VMEM((RB,), jnp.float32),
            pltpu.VMEM_SHARED((AGR,), jnp.float32),
        ],
    )(idx2)


def _zero_agg(buf, agg, s):
    zero = jnp.zeros((16,), jnp.float32)

    def zrow(i, carry):
        for k in range(8):
            buf[i, pl.ds(16 * k, 16)] = zero
        return carry

    lax.fori_loop(0, RB, zrow, None)
    nblk = (NRB - s + NS - 1) // NS

    def zblk(k, carry):
        j = s + k * NS
        pltpu.sync_copy(buf.at[pl.ds(0, RB)], agg.at[pl.ds(j * RB, RB)])
        return carry

    lax.fori_loop(0, nblk, zblk, None)

    @pl.when(s == 0)
    def _():
        pltpu.sync_copy(buf.at[pl.ds(0, 8)], agg.at[pl.ds(NU, 8)])


def _conv_pass(tab, gidx, sidx, buf, agg, out_slot, s):
    _zero_agg(buf, agg, s)
    plsc.subcore_barrier()

    def conv(j, carry):
        pltpu.sync_copy(tab.at[gidx.at[j]], buf)
        pltpu.sync_copy(buf, agg.at[sidx.at[j]], add=True)
        return carry

    lax.fori_loop(0, CPT, conv, None)
    plsc.subcore_barrier()
    nblk = (NRB - s + NS - 1) // NS

    def dump(k, carry):
        j = s + k * NS
        pltpu.sync_copy(agg.at[pl.ds(j * RB, RB)], buf.at[pl.ds(0, RB)])
        pltpu.sync_copy(buf.at[pl.ds(0, RB)], out_slot.at[pl.ds(j * RB, RB)])
        return carry

    lax.fori_loop(0, nblk, dump, None)
    plsc.subcore_barrier()


def _l1_body(hr3, hv3, idx2, agg4, srcv, dstv, buf, agg):
    c = lax.axis_index("c")
    s = lax.axis_index("s")
    pltpu.sync_copy(idx2.at[0, s], srcv)
    pltpu.sync_copy(idx2.at[1, s], dstv)
    for slot, tab3, gidx, sidx in ((0, hr3, srcv, dstv), (1, hv3, dstv, srcv)):
        _conv_pass(tab3.at[c], gidx, sidx, buf, agg, agg4.at[slot, c], s)


def _conv1(hr3, hv3, idx2):
    return pl.kernel(
        _l1_body,
        out_type=jax.ShapeDtypeStruct((2, 2, NU, HF), jnp.float32),
        mesh=_mesh(),
        scratch_types=[
            pltpu.VMEM((CPT, CH), jnp.int32),
            pltpu.VMEM((CPT, CH), jnp.int32),
            pltpu.VMEM((CH, HF), jnp.float32),
            pltpu.VMEM_SHARED((AGR, HF), jnp.float32),
        ],
    )(hr3, hv3, idx2)


def _l2_body(g3, idx2, norm2, b2f, out2, srcv, dstv, buf, nb, bb, agg):
    c = lax.axis_index("c")
    s = lax.axis_index("s")
    pltpu.sync_copy(idx2.at[c, s], srcv)
    pltpu.sync_copy(idx2.at[1 - c, s], dstv)
    pltpu.sync_copy(b2f.at[pl.ds(c * FO, FO)], bb)
    _zero_agg(buf, agg, s)
    plsc.subcore_barrier()

    def conv(j, carry):
        pltpu.sync_copy(g3.at[c].at[srcv.at[j]], buf)
        pltpu.sync_copy(buf, agg.at[dstv.at[j]], add=True)
        return carry

    lax.fori_loop(0, CPT, conv, None)
    plsc.subcore_barrier()
    nblk = (NRB - s + NS - 1) // NS

    def dump(k, carry):
        j = s + k * NS
        pltpu.sync_copy(agg.at[pl.ds(j * RB, RB)], buf.at[pl.ds(0, RB)])
        pltpu.sync_copy(norm2.at[pl.ds((1 - c) * NU + j * RB, RB)], nb)

        def grp(m, carry2):
            nv = nb[pl.ds(16 * m, 16)]
            for r in range(16):
                i = 16 * m + r
                bv = jnp.take(nv, jnp.full((16,), r, jnp.int32))
                for k2 in range(8):
                    sl = pl.ds(16 * k2, 16)
                    buf[i, sl] = buf[i, sl] * bv + bb[sl]
            return carry2

        lax.fori_loop(0, RB // 16, grp, None)
        pltpu.sync_copy(buf.at[pl.ds(0, RB)], out2.at[c, pl.ds(j * RB, RB)])
        return carry

    lax.fori_loop(0, nblk, dump, None)


def _conv2(g3, idx2, norm2, b2f):
    return pl.kernel(
        _l2_body,
        out_type=jax.ShapeDtypeStruct((2, NU, FO), jnp.float32),
        mesh=_mesh(),
        scratch_types=[
            pltpu.VMEM((CPT, CH), jnp.int32),
            pltpu.VMEM((CPT, CH), jnp.int32),
            pltpu.VMEM((CH, FO), jnp.float32),
            pltpu.VMEM((RB,), jnp.float32),
            pltpu.VMEM((FO,), jnp.float32),
            pltpu.VMEM_SHARED((AGR, FO), jnp.float32),
        ],
    )(g3, idx2, norm2, b2f)


_RBLK = 128
_NB = NPAD // _RBLK


def _mm1_body(xu, xi, du, di, wr, wv, hr3, hv3, norm3):
    h = pl.program_id(1)
    nu = lax.rsqrt(jnp.maximum(du[0], 1.0))
    ni = lax.rsqrt(jnp.maximum(di[0], 1.0))
    hr3[...] = jnp.dot(xu[...] * nu, wr[0],
                       preferred_element_type=jnp.float32)[None]
    hv3[...] = jnp.dot(xi[...] * ni, wv[0],
                       preferred_element_type=jnp.float32)[None]
    norm3[...] = jnp.where(h == 0, nu, ni)[None]


def _mm1(xu, xi, deg3, wr, wv):
    bs_x = pl.BlockSpec((_RBLK, FIN), lambda i, h: (i, 0))
    bs_du = pl.BlockSpec((1, _RBLK, 1), lambda i, h: (0, i, 0))
    bs_di = pl.BlockSpec((1, _RBLK, 1), lambda i, h: (1, i, 0))
    bs_w = pl.BlockSpec((1, FIN, HF), lambda i, h: (h, 0, 0))
    bs_h = pl.BlockSpec((1, _RBLK, HF), lambda i, h: (h, i, 0))
    bs_n = pl.BlockSpec((1, _RBLK, 1), lambda i, h: (h, i, 0))
    return pl.pallas_call(
        _mm1_body, grid=(_NB, 2),
        in_specs=[bs_x, bs_x, bs_du, bs_di, bs_w, bs_w],
        out_specs=[bs_h, bs_h, bs_n],
        out_shape=[jax.ShapeDtypeStruct((2, NPAD, HF), jnp.float32)] * 2
        + [jax.ShapeDtypeStruct((2, NU, 1), jnp.float32)],
    )(xu, xi, deg3, deg3, wr, wv)


def _mm2_body(a0, a1, dg, b1, w2, g3):
    n = lax.rsqrt(jnp.maximum(dg[0], 1.0))
    a = jnp.concatenate([a0[0, 0], a1[0, 0]], axis=1)
    t = jnp.maximum(a * n + b1[0], 0.0) * n
    g3[...] = jnp.dot(t, w2[0], preferred_element_type=jnp.float32)[None]


def _mm2(agg4, deg3, b1s, w2s):
    def bs_ak(k):
        return pl.BlockSpec((1, 1, _RBLK, HF),
                            lambda i, r, k=k: (1 - r, k, i, 0))
    bs_d = pl.BlockSpec((1, _RBLK, 1), lambda i, r: (r, i, 0))
    bs_b = pl.BlockSpec((1, 1, FH), lambda i, r: (r, 0, 0))
    bs_w = pl.BlockSpec((1, FH, FO), lambda i, r: (r, 0, 0))
    bs_g = pl.BlockSpec((1, _RBLK, FO), lambda i, r: (r, i, 0))
    return pl.pallas_call(
        _mm2_body, grid=(_NB, 2),
        in_specs=[bs_ak(0), bs_ak(1), bs_d, bs_b, bs_w],
        out_specs=bs_g,
        out_shape=jax.ShapeDtypeStruct((2, NPAD, FO), jnp.float32),
    )(agg4, agg4, deg3, b1s, w2s)


def kernel(x_user, x_item, src_rates, dst_rates,
           W1_rates, b1_rates, W1_rev, b1_rev,
           W2_rates, b2_rates, W2_rev, b2_rev):
    pad = DUMMY + (jnp.arange(NEP - NE, dtype=jnp.int32) % 8)
    srcp = (jnp.concatenate([src_rates, pad])
            .reshape(CPT, NS, CH).transpose(1, 0, 2))
    dstp = (jnp.concatenate([dst_rates, pad])
            .reshape(CPT, NS, CH).transpose(1, 0, 2))
    idx2 = jnp.stack([srcp, dstp])
    deg = _degrees(idx2)
    deg3 = deg.reshape(2, NU, 1)
    w1rh = W1_rates.reshape(FIN, 2, HF).transpose(1, 0, 2)
    w1vh = W1_rev.reshape(FIN, 2, HF).transpose(1, 0, 2)
    hr3, hv3, norm3 = _mm1(x_user, x_item, deg3, w1rh, w1vh)
    norm2 = norm3.reshape(2 * NU)
    agg4 = _conv1(hr3, hv3, idx2)
    b1s = jnp.stack([b1_rev, b1_rates]).reshape(2, 1, FH)
    w2s = jnp.stack([W2_rates, W2_rev])
    g3 = _mm2(agg4, deg3, b1s, w2s)
    b2f = jnp.concatenate([b2_rates, b2_rev])
    h2 = _conv2(g3, idx2, norm2, b2f)
    return (h2[1], h2[0])


# speedup vs baseline: 1.7650x; 1.0758x over previous
import jax
import jax.numpy as jnp
from jax import lax
from jax.experimental import pallas as pl
from jax.experimental.pallas import tpu as pltpu
from jax.experimental.pallas import tpu_sc as plsc

NU = 10000
NE = 160000
FIN = 256
FH = 256
FO = 128
HF = 128

NC = 2
NS = 16
CH = 128
CPT = 80
HCPT = CPT // 2
NEP = NS * CPT * CH
DUMMY = NU
AGR = NU + 8
NPAD = 79 * 128
RB = 80
NRB = NU // RB


def _mesh():
    return plsc.VectorSubcoreMesh(
        core_axis_name="c", subcore_axis_name="s",
        num_cores=NC, num_subcores=NS)


def _deg_body(idx2, deg, idxv, onesv, zb, ob, hist):
    c = lax.axis_index("c")
    s = lax.axis_index("s")
    one = jnp.ones((16,), jnp.float32)
    zero = jnp.zeros((16,), jnp.float32)
    for k in range(CH // 16):
        onesv[pl.ds(16 * k, 16)] = one
    for k in range(RB // 16):
        zb[pl.ds(16 * k, 16)] = zero
    nblk = (NRB - s + NS - 1) // NS

    def zblk(k, carry):
        j = s + k * NS
        pltpu.sync_copy(zb, hist.at[pl.ds(j * RB, RB)])
        return carry

    lax.fori_loop(0, nblk, zblk, None)

    @pl.when(s == 0)
    def _():
        pltpu.sync_copy(zb.at[pl.ds(0, 8)], hist.at[pl.ds(NU, 8)])

    plsc.subcore_barrier()
    pltpu.sync_copy(idx2.at[c, s], idxv)

    def acc(j, carry):
        pltpu.sync_copy(onesv, hist.at[idxv.at[j]], add=True)
        return carry

    lax.fori_loop(0, CPT, acc, None)
    plsc.subcore_barrier()

    def dump(k, carry):
        j = s + k * NS
        pltpu.sync_copy(hist.at[pl.ds(j * RB, RB)], ob)
        pltpu.sync_copy(ob, deg.at[pl.ds(c * NU + j * RB, RB)])
        return carry

    lax.fori_loop(0, nblk, dump, None)


def _degrees(idx2):
    return pl.kernel(
        _deg_body,
        out_type=jax.ShapeDtypeStruct((2 * NU,), jnp.float32),
        mesh=_mesh(),
        scratch_types=[
            pltpu.VMEM((CPT, CH), jnp.int32),
            pltpu.VMEM((CH,), jnp.float32),
            pltpu.VMEM((RB,), jnp.float32),
            pltpu.VMEM((RB,), jnp.float32),
            pltpu.VMEM_SHARED((AGR,), jnp.float32),
        ],
    )(idx2)


def _zero_agg(buf, agg, s):
    zero = jnp.zeros((16,), jnp.float32)

    def zrow(i, carry):
        for k in range(8):
            buf[i, pl.ds(16 * k, 16)] = zero
        return carry

    lax.fori_loop(0, RB, zrow, None)
    nblk = (NRB - s + NS - 1) // NS

    def zblk(k, carry):
        j = s + k * NS
        pltpu.sync_copy(buf.at[pl.ds(0, RB)], agg.at[pl.ds(j * RB, RB)])
        return carry

    lax.fori_loop(0, nblk, zblk, None)

    @pl.when(s == 0)
    def _():
        pltpu.sync_copy(buf.at[pl.ds(0, 8)], agg.at[pl.ds(NU, 8)])


def _conv_loop(tab, gview, sview, gidx, sidx, b0, b1, gs0, gs1, ss0, ss1, agg):
    npairs = HCPT // 2
    for half in (0, 1):
        pltpu.sync_copy(gview.at[pl.ds(half * HCPT, HCPT)], gidx)
        pltpu.sync_copy(sview.at[pl.ds(half * HCPT, HCPT)], sidx)
        pltpu.async_copy(tab.at[gidx.at[0]], b0, gs0)
        pltpu.async_copy(tab.at[gidx.at[1]], b1, gs1)

        def pair(k, carry):
            j0 = 2 * k
            j1 = j0 + 1
            pltpu.make_async_copy(tab.at[gidx.at[j0]], b0, gs0).wait()
            sd0 = pltpu.async_copy(b0, agg.at[sidx.at[j0]], ss0, add=True)
            pltpu.make_async_copy(tab.at[gidx.at[j1]], b1, gs1).wait()
            sd1 = pltpu.async_copy(b1, agg.at[sidx.at[j1]], ss1, add=True)
            sd0.wait()

            @pl.when(k < npairs - 1)
            def _():
                pltpu.async_copy(tab.at[gidx.at[j0 + 2]], b0, gs0)

            sd1.wait()

            @pl.when(k < npairs - 1)
            def _():
                pltpu.async_copy(tab.at[gidx.at[j1 + 2]], b1, gs1)

            return carry

        lax.fori_loop(0, npairs, pair, None)


def _conv_pass(tab, gview, sview, gidx, sidx, b0, b1, gs0, gs1, ss0, ss1,
               agg, out_slot, s):
    _zero_agg(b0, agg, s)
    plsc.subcore_barrier()
    _conv_loop(tab, gview, sview, gidx, sidx, b0, b1, gs0, gs1, ss0, ss1, agg)
    plsc.subcore_barrier()
    nblk = (NRB - s + NS - 1) // NS

    def dump(k, carry):
        j = s + k * NS
        pltpu.sync_copy(agg.at[pl.ds(j * RB, RB)], b0.at[pl.ds(0, RB)])
        pltpu.sync_copy(b0.at[pl.ds(0, RB)], out_slot.at[pl.ds(j * RB, RB)])
        return carry

    lax.fori_loop(0, nblk, dump, None)
    plsc.subcore_barrier()


_CONV_SCRATCH = [
    pltpu.VMEM((HCPT, CH), jnp.int32),
    pltpu.VMEM((HCPT, CH), jnp.int32),
    pltpu.VMEM((CH, HF), jnp.float32),
    pltpu.VMEM((CH, HF), jnp.float32),
    pltpu.SemaphoreType.DMA,
    pltpu.SemaphoreType.DMA,
    pltpu.SemaphoreType.DMA,
    pltpu.SemaphoreType.DMA,
]


def _l1_body(hr3, hv3, idx2, agg4,
             gidx, sidx, b0, b1, gs0, gs1, ss0, ss1, agg):
    c = lax.axis_index("c")
    s = lax.axis_index("s")
    for slot, tab3, g in ((0, hr3, 0), (1, hv3, 1)):
        _conv_pass(tab3.at[c], idx2.at[g, s], idx2.at[1 - g, s],
                   gidx, sidx, b0, b1, gs0, gs1, ss0, ss1,
                   agg, agg4.at[slot, c], s)


def _conv1(hr3, hv3, idx2):
    return pl.kernel(
        _l1_body,
        out_type=jax.ShapeDtypeStruct((2, 2, NU, HF), jnp.float32),
        mesh=_mesh(),
        scratch_types=_CONV_SCRATCH
        + [pltpu.VMEM_SHARED((AGR, HF), jnp.float32)],
    )(hr3, hv3, idx2)


def _l2_body(g3, idx2, norm2, b2f, out2,
             gidx, sidx, b0, b1, gs0, gs1, ss0, ss1, nb, bb, agg):
    c = lax.axis_index("c")
    s = lax.axis_index("s")
    pltpu.sync_copy(b2f.at[pl.ds(c * FO, FO)], bb)
    _zero_agg(b0, agg, s)
    plsc.subcore_barrier()
    _conv_loop(g3.at[c], idx2.at[c, s], idx2.at[1 - c, s],
               gidx, sidx, b0, b1, gs0, gs1, ss0, ss1, agg)
    plsc.subcore_barrier()
    nblk = (NRB - s + NS - 1) // NS

    def dump(k, carry):
        j = s + k * NS
        pltpu.sync_copy(agg.at[pl.ds(j * RB, RB)], b0.at[pl.ds(0, RB)])
        pltpu.sync_copy(norm2.at[pl.ds((1 - c) * NU + j * RB, RB)], nb)

        def grp(m, carry2):
            nv = nb[pl.ds(16 * m, 16)]
            for r in range(16):
                i = 16 * m + r
                bv = jnp.take(nv, jnp.full((16,), r, jnp.int32))
                for k2 in range(8):
                    sl = pl.ds(16 * k2, 16)
                    b0[i, sl] = b0[i, sl] * bv + bb[sl]
            return carry2

        lax.fori_loop(0, RB // 16, grp, None)
        pltpu.sync_copy(b0.at[pl.ds(0, RB)], out2.at[c, pl.ds(j * RB, RB)])
        return carry

    lax.fori_loop(0, nblk, dump, None)


def _conv2(g3, idx2, norm2, b2f):
    return pl.kernel(
        _l2_body,
        out_type=jax.ShapeDtypeStruct((2, NU, FO), jnp.float32),
        mesh=_mesh(),
        scratch_types=_CONV_SCRATCH + [
            pltpu.VMEM((RB,), jnp.float32),
            pltpu.VMEM((FO,), jnp.float32),
            pltpu.VMEM_SHARED((AGR, FO), jnp.float32),
        ],
    )(g3, idx2, norm2, b2f)


_RBLK = 128
_NB = NPAD // _RBLK


def _mm1_body(xu, xi, du, di, wr, wv, hr3, hv3, norm3):
    h = pl.program_id(1)
    nu = lax.rsqrt(jnp.maximum(du[0], 1.0))
    ni = lax.rsqrt(jnp.maximum(di[0], 1.0))
    hr3[...] = jnp.dot(xu[...] * nu, wr[0],
                       preferred_element_type=jnp.float32)[None]
    hv3[...] = jnp.dot(xi[...] * ni, wv[0],
                       preferred_element_type=jnp.float32)[None]
    norm3[...] = jnp.where(h == 0, nu, ni)[None]


def _mm1(xu, xi, deg3, wr, wv):
    bs_x = pl.BlockSpec((_RBLK, FIN), lambda i, h: (i, 0))
    bs_du = pl.BlockSpec((1, _RBLK, 1), lambda i, h: (0, i, 0))
    bs_di = pl.BlockSpec((1, _RBLK, 1), lambda i, h: (1, i, 0))
    bs_w = pl.BlockSpec((1, FIN, HF), lambda i, h: (h, 0, 0))
    bs_h = pl.BlockSpec((1, _RBLK, HF), lambda i, h: (h, i, 0))
    bs_n = pl.BlockSpec((1, _RBLK, 1), lambda i, h: (h, i, 0))
    return pl.pallas_call(
        _mm1_body, grid=(_NB, 2),
        in_specs=[bs_x, bs_x, bs_du, bs_di, bs_w, bs_w],
        out_specs=[bs_h, bs_h, bs_n],
        out_shape=[jax.ShapeDtypeStruct((2, NPAD, HF), jnp.float32)] * 2
        + [jax.ShapeDtypeStruct((2, NU, 1), jnp.float32)],
    )(xu, xi, deg3, deg3, wr, wv)


def _mm2_body(a0, a1, dg, b1, w2, g3):
    n = lax.rsqrt(jnp.maximum(dg[0], 1.0))
    a = jnp.concatenate([a0[0, 0], a1[0, 0]], axis=1)
    t = jnp.maximum(a * n + b1[0], 0.0) * n
    g3[...] = jnp.dot(t, w2[0], preferred_element_type=jnp.float32)[None]


def _mm2(agg4, deg3, b1s, w2s):
    def bs_ak(k):
        return pl.BlockSpec((1, 1, _RBLK, HF),
                            lambda i, r, k=k: (1 - r, k, i, 0))
    bs_d = pl.BlockSpec((1, _RBLK, 1), lambda i, r: (r, i, 0))
    bs_b = pl.BlockSpec((1, 1, FH), lambda i, r: (r, 0, 0))
    bs_w = pl.BlockSpec((1, FH, FO), lambda i, r: (r, 0, 0))
    bs_g = pl.BlockSpec((1, _RBLK, FO), lambda i, r: (r, i, 0))
    return pl.pallas_call(
        _mm2_body, grid=(_NB, 2),
        in_specs=[bs_ak(0), bs_ak(1), bs_d, bs_b, bs_w],
        out_specs=bs_g,
        out_shape=jax.ShapeDtypeStruct((2, NPAD, FO), jnp.float32),
    )(agg4, agg4, deg3, b1s, w2s)


def kernel(x_user, x_item, src_rates, dst_rates,
           W1_rates, b1_rates, W1_rev, b1_rev,
           W2_rates, b2_rates, W2_rev, b2_rev):
    pad = DUMMY + (jnp.arange(NEP - NE, dtype=jnp.int32) % 8)
    srcp = (jnp.concatenate([src_rates, pad])
            .reshape(CPT, NS, CH).transpose(1, 0, 2))
    dstp = (jnp.concatenate([dst_rates, pad])
            .reshape(CPT, NS, CH).transpose(1, 0, 2))
    idx2 = jnp.stack([srcp, dstp])
    deg = _degrees(idx2)
    deg3 = deg.reshape(2, NU, 1)
    w1rh = W1_rates.reshape(FIN, 2, HF).transpose(1, 0, 2)
    w1vh = W1_rev.reshape(FIN, 2, HF).transpose(1, 0, 2)
    hr3, hv3, norm3 = _mm1(x_user, x_item, deg3, w1rh, w1vh)
    norm2 = norm3.reshape(2 * NU)
    agg4 = _conv1(hr3, hv3, idx2)
    b1s = jnp.stack([b1_rev, b1_rates]).reshape(2, 1, FH)
    w2s = jnp.stack([W2_rates, W2_rev])
    g3 = _mm2(agg4, deg3, b1s, w2s)
    b2f = jnp.concatenate([b2_rates, b2_rev])
    h2 = _conv2(g3, idx2, norm2, b2f)
    return (h2[1], h2[0])


# speedup vs baseline: 1.9239x; 1.0900x over previous
import jax
import jax.numpy as jnp
from jax import lax
from jax.experimental import pallas as pl
from jax.experimental.pallas import tpu as pltpu
from jax.experimental.pallas import tpu_sc as plsc

NU = 10000
NE = 160000
FIN = 256
FH = 256
FO = 128
HF = 128

NC = 2
NS = 16
CH = 64
NBUF = 4
CPT = 160
QCPT = CPT // 4
NEP = NS * CPT * CH
DUMMY = NU
AGR = NU + 8
NPAD = 79 * 128
RB = 64
NFB = NU // RB
TAIL = NU - NFB * RB
TS = NFB % NS
DRB = 80
DNRB = NU // DRB


def _mesh():
    return plsc.VectorSubcoreMesh(
        core_axis_name="c", subcore_axis_name="s",
        num_cores=NC, num_subcores=NS)


def _deg_body(idx2, deg, idxv, onesv, zb, ob, hist):
    c = lax.axis_index("c")
    s = lax.axis_index("s")
    one = jnp.ones((16,), jnp.float32)
    zero = jnp.zeros((16,), jnp.float32)
    for k in range(CH // 16):
        onesv[pl.ds(16 * k, 16)] = one
    for k in range(DRB // 16):
        zb[pl.ds(16 * k, 16)] = zero
    nblk = (DNRB - s + NS - 1) // NS

    def zblk(k, carry):
        j = s + k * NS
        pltpu.sync_copy(zb, hist.at[pl.ds(j * DRB, DRB)])
        return carry

    lax.fori_loop(0, nblk, zblk, None)

    @pl.when(s == 0)
    def _():
        pltpu.sync_copy(zb.at[pl.ds(0, 8)], hist.at[pl.ds(NU, 8)])

    plsc.subcore_barrier()
    pltpu.sync_copy(idx2.at[c, s], idxv)

    def acc(j, carry):
        pltpu.sync_copy(onesv, hist.at[idxv.at[j]], add=True)
        return carry

    lax.fori_loop(0, CPT, acc, None)
    plsc.subcore_barrier()

    def dump(k, carry):
        j = s + k * NS
        pltpu.sync_copy(hist.at[pl.ds(j * DRB, DRB)], ob)
        pltpu.sync_copy(ob, deg.at[pl.ds(c * NU + j * DRB, DRB)])
        return carry

    lax.fori_loop(0, nblk, dump, None)


def _degrees(idx2):
    return pl.kernel(
        _deg_body,
        out_type=jax.ShapeDtypeStruct((2 * NU,), jnp.float32),
        mesh=_mesh(),
        scratch_types=[
            pltpu.VMEM((CPT, CH), jnp.int32),
            pltpu.VMEM((CH,), jnp.float32),
            pltpu.VMEM((DRB,), jnp.float32),
            pltpu.VMEM((DRB,), jnp.float32),
            pltpu.VMEM_SHARED((AGR,), jnp.float32),
        ],
    )(idx2)


def _zero_agg(buf, agg, s):
    zero = jnp.zeros((16,), jnp.float32)

    def zrow(i, carry):
        for k in range(8):
            buf[i, pl.ds(16 * k, 16)] = zero
        return carry

    lax.fori_loop(0, RB, zrow, None)
    nblk = (NFB - s + NS - 1) // NS

    def zblk(k, carry):
        j = s + k * NS
        pltpu.sync_copy(buf.at[pl.ds(0, RB)], agg.at[pl.ds(j * RB, RB)])
        return carry

    lax.fori_loop(0, nblk, zblk, None)

    @pl.when(s == TS)
    def _():
        pltpu.sync_copy(buf.at[pl.ds(0, TAIL + 8)],
                        agg.at[pl.ds(NFB * RB, TAIL + 8)])


def _conv_loop(tab, gview, sview, gidx, sidx, bufs, gsems, ssems, agg):
    nquad = QCPT // NBUF
    for q in range(4):
        pltpu.sync_copy(gview.at[pl.ds(q * QCPT, QCPT)], gidx)
        pltpu.sync_copy(sview.at[pl.ds(q * QCPT, QCPT)], sidx)
        for b in range(NBUF):
            pltpu.async_copy(tab.at[gidx.at[b]], bufs[b], gsems[b])

        def quad(k, carry):
            j = NBUF * k
            sds = []
            for b in range(NBUF):
                pltpu.make_async_copy(tab.at[gidx.at[j + b]],
                                      bufs[b], gsems[b]).wait()
                sds.append(pltpu.async_copy(
                    bufs[b], agg.at[sidx.at[j + b]], ssems[b], add=True))
            for b in range(NBUF):
                sds[b].wait()

                @pl.when(k < nquad - 1)
                def _(b=b):
                    pltpu.async_copy(tab.at[gidx.at[j + b + NBUF]],
                                     bufs[b], gsems[b])

            return carry

        lax.fori_loop(0, nquad, quad, None)


def _conv_pass(tab, gview, sview, gidx, sidx, bufs, gsems, ssems,
               agg, out_slot, s):
    b0 = bufs[0]
    _zero_agg(b0, agg, s)
    plsc.subcore_barrier()
    _conv_loop(tab, gview, sview, gidx, sidx, bufs, gsems, ssems, agg)
    plsc.subcore_barrier()
    nblk = (NFB - s + NS - 1) // NS

    def dump(k, carry):
        j = s + k * NS
        pltpu.sync_copy(agg.at[pl.ds(j * RB, RB)], b0.at[pl.ds(0, RB)])
        pltpu.sync_copy(b0.at[pl.ds(0, RB)], out_slot.at[pl.ds(j * RB, RB)])
        return carry

    lax.fori_loop(0, nblk, dump, None)

    @pl.when(s == TS)
    def _():
        pltpu.sync_copy(agg.at[pl.ds(NFB * RB, TAIL)], b0.at[pl.ds(0, TAIL)])
        pltpu.sync_copy(b0.at[pl.ds(0, TAIL)],
                        out_slot.at[pl.ds(NFB * RB, TAIL)])

    plsc.subcore_barrier()


_CONV_SCRATCH = [
    pltpu.VMEM((QCPT, CH), jnp.int32),
    pltpu.VMEM((QCPT, CH), jnp.int32),
] + [pltpu.VMEM((CH, HF), jnp.float32)] * NBUF \
  + [pltpu.SemaphoreType.DMA] * (2 * NBUF)


def _l1_body(hr3, hv3, idx2, agg4, gidx, sidx, *rest):
    bufs = rest[:NBUF]
    gsems = rest[NBUF:2 * NBUF]
    ssems = rest[2 * NBUF:3 * NBUF]
    agg = rest[3 * NBUF]
    c = lax.axis_index("c")
    s = lax.axis_index("s")
    for slot, tab3, g in ((0, hr3, 0), (1, hv3, 1)):
        _conv_pass(tab3.at[c], idx2.at[g, s], idx2.at[1 - g, s],
                   gidx, sidx, bufs, gsems, ssems,
                   agg, agg4.at[slot, c], s)


def _conv1(hr3, hv3, idx2):
    return pl.kernel(
        _l1_body,
        out_type=jax.ShapeDtypeStruct((2, 2, NU, HF), jnp.float32),
        mesh=_mesh(),
        scratch_types=_CONV_SCRATCH
        + [pltpu.VMEM_SHARED((AGR, HF), jnp.float32)],
    )(hr3, hv3, idx2)


def _epilogue_rows(b0, nb, bb, nrows):
    def grp(m, carry):
        nv = nb[pl.ds(16 * m, 16)]
        for r in range(16):
            i = 16 * m + r
            bv = jnp.take(nv, jnp.full((16,), r, jnp.int32))
            for k2 in range(8):
                sl = pl.ds(16 * k2, 16)
                b0[i, sl] = b0[i, sl] * bv + bb[sl]
        return carry

    lax.fori_loop(0, nrows // 16, grp, None)


def _l2_body(g3, idx2, norm2, b2f, out2, gidx, sidx, *rest):
    bufs = rest[:NBUF]
    gsems = rest[NBUF:2 * NBUF]
    ssems = rest[2 * NBUF:3 * NBUF]
    nb, bb, agg = rest[3 * NBUF:]
    b0 = bufs[0]
    c = lax.axis_index("c")
    s = lax.axis_index("s")
    pltpu.sync_copy(b2f.at[pl.ds(c * FO, FO)], bb)
    _zero_agg(b0, agg, s)
    plsc.subcore_barrier()
    _conv_loop(g3.at[c], idx2.at[c, s], idx2.at[1 - c, s],
               gidx, sidx, bufs, gsems, ssems, agg)
    plsc.subcore_barrier()
    nblk = (NFB - s + NS - 1) // NS

    def dump(k, carry):
        j = s + k * NS
        pltpu.sync_copy(agg.at[pl.ds(j * RB, RB)], b0.at[pl.ds(0, RB)])
        pltpu.sync_copy(norm2.at[pl.ds((1 - c) * NU + j * RB, RB)], nb)
        _epilogue_rows(b0, nb, bb, RB)
        pltpu.sync_copy(b0.at[pl.ds(0, RB)], out2.at[c, pl.ds(j * RB, RB)])
        return carry

    lax.fori_loop(0, nblk, dump, None)

    @pl.when(s == TS)
    def _():
        base = NFB * RB
        pltpu.sync_copy(agg.at[pl.ds(base, TAIL)], b0.at[pl.ds(0, TAIL)])
        pltpu.sync_copy(norm2.at[pl.ds((1 - c) * NU + base, TAIL)],
                        nb.at[pl.ds(0, TAIL)])
        _epilogue_rows(b0, nb, bb, TAIL)
        pltpu.sync_copy(b0.at[pl.ds(0, TAIL)], out2.at[c, pl.ds(base, TAIL)])


def _conv2(g3, idx2, norm2, b2f):
    return pl.kernel(
        _l2_body,
        out_type=jax.ShapeDtypeStruct((2, NU, FO), jnp.float32),
        mesh=_mesh(),
        scratch_types=_CONV_SCRATCH + [
            pltpu.VMEM((RB,), jnp.float32),
            pltpu.VMEM((FO,), jnp.float32),
            pltpu.VMEM_SHARED((AGR, FO), jnp.float32),
        ],
    )(g3, idx2, norm2, b2f)


_RBLK = 128
_NB = NPAD // _RBLK


def _mm1_body(xu, xi, du, di, wr, wv, hr3, hv3, norm3):
    h = pl.program_id(1)
    nu = lax.rsqrt(jnp.maximum(du[0], 1.0))
    ni = lax.rsqrt(jnp.maximum(di[0], 1.0))
    hr3[...] = jnp.dot(xu[...] * nu, wr[0],
                       preferred_element_type=jnp.float32)[None]
    hv3[...] = jnp.dot(xi[...] * ni, wv[0],
                       preferred_element_type=jnp.float32)[None]
    norm3[...] = jnp.where(h == 0, nu, ni)[None]


def _mm1(xu, xi, deg3, wr, wv):
    bs_x = pl.BlockSpec((_RBLK, FIN), lambda i, h: (i, 0))
    bs_du = pl.BlockSpec((1, _RBLK, 1), lambda i, h: (0, i, 0))
    bs_di = pl.BlockSpec((1, _RBLK, 1), lambda i, h: (1, i, 0))
    bs_w = pl.BlockSpec((1, FIN, HF), lambda i, h: (h, 0, 0))
    bs_h = pl.BlockSpec((1, _RBLK, HF), lambda i, h: (h, i, 0))
    bs_n = pl.BlockSpec((1, _RBLK, 1), lambda i, h: (h, i, 0))
    return pl.pallas_call(
        _mm1_body, grid=(_NB, 2),
        in_specs=[bs_x, bs_x, bs_du, bs_di, bs_w, bs_w],
        out_specs=[bs_h, bs_h, bs_n],
        out_shape=[jax.ShapeDtypeStruct((2, NPAD, HF), jnp.float32)] * 2
        + [jax.ShapeDtypeStruct((2, NU, 1), jnp.float32)],
    )(xu, xi, deg3, deg3, wr, wv)


def _mm2_body(a0, a1, dg, b1, w2, g3):
    n = lax.rsqrt(jnp.maximum(dg[0], 1.0))
    a = jnp.concatenate([a0[0, 0], a1[0, 0]], axis=1)
    t = jnp.maximum(a * n + b1[0], 0.0) * n
    g3[...] = jnp.dot(t, w2[0], preferred_element_type=jnp.float32)[None]


def _mm2(agg4, deg3, b1s, w2s):
    def bs_ak(k):
        return pl.BlockSpec((1, 1, _RBLK, HF),
                            lambda i, r, k=k: (1 - r, k, i, 0))
    bs_d = pl.BlockSpec((1, _RBLK, 1), lambda i, r: (r, i, 0))
    bs_b = pl.BlockSpec((1, 1, FH), lambda i, r: (r, 0, 0))
    bs_w = pl.BlockSpec((1, FH, FO), lambda i, r: (r, 0, 0))
    bs_g = pl.BlockSpec((1, _RBLK, FO), lambda i, r: (r, i, 0))
    return pl.pallas_call(
        _mm2_body, grid=(_NB, 2),
        in_specs=[bs_ak(0), bs_ak(1), bs_d, bs_b, bs_w],
        out_specs=bs_g,
        out_shape=jax.ShapeDtypeStruct((2, NPAD, FO), jnp.float32),
    )(agg4, agg4, deg3, b1s, w2s)


def kernel(x_user, x_item, src_rates, dst_rates,
           W1_rates, b1_rates, W1_rev, b1_rev,
           W2_rates, b2_rates, W2_rev, b2_rev):
    pad = DUMMY + (jnp.arange(NEP - NE, dtype=jnp.int32) % 8)
    srcp = (jnp.concatenate([src_rates, pad])
            .reshape(CPT, NS, CH).transpose(1, 0, 2))
    dstp = (jnp.concatenate([dst_rates, pad])
            .reshape(CPT, NS, CH).transpose(1, 0, 2))
    idx2 = jnp.stack([srcp, dstp])
    deg = _degrees(idx2)
    deg3 = deg.reshape(2, NU, 1)
    w1rh = W1_rates.reshape(FIN, 2, HF).transpose(1, 0, 2)
    w1vh = W1_rev.reshape(FIN, 2, HF).transpose(1, 0, 2)
    hr3, hv3, norm3 = _mm1(x_user, x_item, deg3, w1rh, w1vh)
    norm2 = norm3.reshape(2 * NU)
    agg4 = _conv1(hr3, hv3, idx2)
    b1s = jnp.stack([b1_rev, b1_rates]).reshape(2, 1, FH)
    w2s = jnp.stack([W2_rates, W2_rev])
    g3 = _mm2(agg4, deg3, b1s, w2s)
    b2f = jnp.concatenate([b2_rates, b2_rev])
    h2 = _conv2(g3, idx2, norm2, b2f)
    return (h2[1], h2[0])
